# baseline (speedup 1.0000x reference)
import functools

import jax
import jax.numpy as jnp
from jax import lax
from jax.experimental import pallas as pl
from jax.experimental.pallas import tpu as pltpu
from jax.experimental.pallas import tpu_sc as plsc

_CH = 128
_SUBS = 8
_SGE = _CH * _SUBS
_FIFO = 2 * _SGE
_LANES = 16
_G16 = _SGE // _LANES


@functools.lru_cache(maxsize=None)
def _make_sc_kernel(N, E, C):
    info = plsc.get_sparse_core_info()
    NC, NS = info.num_cores, info.num_subcores
    assert E % _SGE == 0 and C % _LANES == 0 and N % NC == 0
    H = N // NC
    assert H % 8 == 0
    hblk = (H + _CH - 1) // _CH
    h_pad = hblk * _CH
    xfull, xrem = divmod(H, _CH)
    nblk = E // _SGE
    qb, rb = divmod(nblk, NS)
    groups = C // _LANES
    tload = (hblk + NS - 1) // NS

    mesh = plsc.VectorSubcoreMesh(core_axis_name="c", subcore_axis_name="s")

    @functools.partial(
        pl.kernel,
        out_type=jax.ShapeDtypeStruct((NC, h_pad, C), jnp.float32),
        mesh=mesh,
        scratch_types=[
            pltpu.VMEM((_CH, C), jnp.float32),
            pltpu.VMEM((_CH, C), jnp.float32),
            pltpu.VMEM((3, _SUBS, _CH), jnp.int32),
            pltpu.VMEM((_FIFO,), jnp.int32),
            pltpu.VMEM((_FIFO,), jnp.int32),
            pltpu.VMEM((_FIFO,), jnp.int32),
            pltpu.VMEM((_FIFO,), jnp.int32),
            pltpu.VMEM((_FIFO,), jnp.int32),
            pltpu.VMEM((_FIFO,), jnp.int32),
            pltpu.VMEM((_CH,), jnp.int32),
            pltpu.VMEM((_CH,), jnp.int32),
            pltpu.VMEM_SHARED((H, C), jnp.float32),
            pltpu.VMEM_SHARED((h_pad, C), jnp.float32),
            pltpu.SemaphoreType.DMA,
            pltpu.SemaphoreType.DMA,
            pltpu.SemaphoreType.DMA,
            pltpu.SemaphoreType.DMA,
            pltpu.SemaphoreType.DMA,
        ],
        compiler_params=pltpu.CompilerParams(needs_layout_passes=False),
    )
    def sc_kernel(x_hbm, comb_hbm, out_hbm,
                  rows0, rows1, comb_buf,
                  fli, flj, flw, fhi, fhj, fhw, jrow0, jrow1,
                  x_sh, acc,
                  isem, gsem0, gsem1, ssem0, ssem1):
        cid = lax.axis_index("c")
        sid = lax.axis_index("s")
        lo = cid * H

        zero = jnp.zeros((_LANES,), jnp.float32)

        @plsc.parallel_loop(0, _CH, unroll=8)
        def zero_rows(rr):
            for g in range(groups):
                rows0[rr, pl.ds(g * _LANES, _LANES)] = zero

        for t in range(tload):
            b = sid + NS * t

            @pl.when(b < hblk)
            def _():
                pltpu.sync_copy(rows0, acc.at[pl.ds(b * _CH, _CH)])

            @pl.when(b < xfull)
            def _():
                pltpu.sync_copy(x_hbm.at[pl.ds(b * _CH, _CH)],
                                x_sh.at[pl.ds(b * _CH, _CH)])

            if xrem:
                @pl.when(b == xfull)
                def _():
                    pltpu.sync_copy(x_hbm.at[pl.ds(xfull * _CH, xrem)],
                                    x_sh.at[pl.ds(xfull * _CH, xrem)])
        plsc.subcore_barrier()

        rows = (rows0, rows1)
        jrows = (jrow0, jrow1)
        gsems = (gsem0, gsem1)
        ssems = (ssem0, ssem1)

        def process_1024(xsrc, fi, fj, fw):
            gd = [None] * _SUBS
            sd = [None] * _SUBS
            gd[0] = pltpu.async_copy(
                xsrc.at[fi.at[pl.ds(0, _CH)]], rows[0], gsems[0])
            for sub in range(_SUBS):
                b = sub & 1
                if sub + 1 < _SUBS:
                    if sub >= 1:
                        sd[sub - 1].wait()
                    gd[sub + 1] = pltpu.async_copy(
                        xsrc.at[fi.at[pl.ds((sub + 1) * _CH, _CH)]],
                        rows[1 - b], gsems[1 - b])
                gd[sub].wait()

                @plsc.parallel_loop(0, _CH, unroll=8)
                def mul_body(e, _rows=rows[b], _sub=sub):
                    w16 = plsc.bitcast(
                        plsc.load_gather(
                            fw, [jnp.full((_LANES,), _sub * _CH, jnp.int32)
                                 + e]),
                        jnp.float32)
                    for g in range(groups):
                        sl = pl.ds(g * _LANES, _LANES)
                        _rows[e, sl] = _rows[e, sl] * w16

                for g in range(_CH // _LANES):
                    jrows[b][pl.ds(g * _LANES, _LANES)] = (
                        fj[pl.ds(sub * _CH + g * _LANES, _LANES)])
                sd[sub] = pltpu.async_copy(
                    rows[b], acc.at[jrows[b]], ssems[b], add=True)
            sd[_SUBS - 2].wait()
            sd[_SUBS - 1].wait()

        bstart = sid * qb + jnp.minimum(sid, rb)
        bcnt = qb + jnp.where(sid < rb, 1, 0)

        izero = jnp.zeros((_LANES,), jnp.int32)

        def blk_body(blk, pos):
            pos_lo, pos_hi = pos
            is_flush = blk >= bcnt
            blk_eff = bstart + jnp.minimum(blk, bcnt - 1)
            pltpu.async_copy(comb_hbm.at[blk_eff], comb_buf, isem).wait()

            def compact(pos2):
                def grp_body(k, p):
                    p_lo, p_hi = p
                    srow = k // (_CH // _LANES)
                    soff = (k % (_CH // _LANES)) * _LANES
                    iv = comb_buf[0, srow, pl.ds(soff, _LANES)]
                    jv = comb_buf[1, srow, pl.ds(soff, _LANES)]
                    wv = comb_buf[2, srow, pl.ds(soff, _LANES)]
                    in_half = (jv >= lo) & (jv < lo + H)
                    is_src_lo = iv < H
                    m_lo = in_half & is_src_lo
                    m_hi = in_half & (~is_src_lo)
                    jl = jv - lo
                    plsc.store_compressed(fli.at[pl.ds(p_lo, _LANES)],
                                          iv, mask=m_lo)
                    plsc.store_compressed(flj.at[pl.ds(p_lo, _LANES)],
                                          jl, mask=m_lo)
                    plsc.store_compressed(flw.at[pl.ds(p_lo, _LANES)],
                                          wv, mask=m_lo)
                    plsc.store_compressed(fhi.at[pl.ds(p_hi, _LANES)],
                                          iv, mask=m_hi)
                    plsc.store_compressed(fhj.at[pl.ds(p_hi, _LANES)],
                                          jl, mask=m_hi)
                    plsc.store_compressed(fhw.at[pl.ds(p_hi, _LANES)],
                                          wv, mask=m_hi)
                    c_lo = jnp.max(plsc.all_reduce_population_count(m_lo))
                    c_hi = jnp.max(plsc.all_reduce_population_count(m_hi))
                    return (p_lo + c_lo, p_hi + c_hi)

                return lax.fori_loop(0, _G16, grp_body, pos2)

            pos_lo, pos_hi = lax.cond(
                is_flush, lambda p: p, compact, (pos_lo, pos_hi))

            @pl.when(is_flush)
            def _():
                for t in range(_SGE // _LANES):
                    fli[pl.ds(pos_lo + t * _LANES, _LANES)] = izero
                    flj[pl.ds(pos_lo + t * _LANES, _LANES)] = izero
                    flw[pl.ds(pos_lo + t * _LANES, _LANES)] = izero
                    fhi[pl.ds(pos_hi + t * _LANES, _LANES)] = izero
                    fhj[pl.ds(pos_hi + t * _LANES, _LANES)] = izero
                    fhw[pl.ds(pos_hi + t * _LANES, _LANES)] = izero

            do_lo = (pos_lo >= _SGE) | (is_flush & (pos_lo > 0))
            do_hi = (pos_hi >= _SGE) | (is_flush & (pos_hi > 0))

            @pl.when(do_lo)
            def _():
                process_1024(x_sh, fli, flj, flw)
                nmv = (jnp.maximum(pos_lo - _SGE, 0) + _LANES - 1) // _LANES

                def mv(t, _):
                    o = t * _LANES
                    fli[pl.ds(o, _LANES)] = fli[pl.ds(_SGE + o, _LANES)]
                    flj[pl.ds(o, _LANES)] = flj[pl.ds(_SGE + o, _LANES)]
                    flw[pl.ds(o, _LANES)] = flw[pl.ds(_SGE + o, _LANES)]
                    return 0

                lax.fori_loop(0, nmv, mv, 0)

            @pl.when(do_hi)
            def _():
                process_1024(x_hbm, fhi, fhj, fhw)
                nmv = (jnp.maximum(pos_hi - _SGE, 0) + _LANES - 1) // _LANES

                def mv(t, _):
                    o = t * _LANES
                    fhi[pl.ds(o, _LANES)] = fhi[pl.ds(_SGE + o, _LANES)]
                    fhj[pl.ds(o, _LANES)] = fhj[pl.ds(_SGE + o, _LANES)]
                    fhw[pl.ds(o, _LANES)] = fhw[pl.ds(_SGE + o, _LANES)]
                    return 0

                lax.fori_loop(0, nmv, mv, 0)

            pos_lo = jnp.where(do_lo, jnp.maximum(pos_lo - _SGE, 0), pos_lo)
            pos_hi = jnp.where(do_hi, jnp.maximum(pos_hi - _SGE, 0), pos_hi)
            return (pos_lo, pos_hi)

        lax.fori_loop(0, bcnt + 1, blk_body,
                      (jnp.int32(0), jnp.int32(0)))
        plsc.subcore_barrier()

        for t in range(tload):
            b = sid + NS * t

            @pl.when(b < hblk)
            def _():
                pltpu.sync_copy(acc.at[pl.ds(b * _CH, _CH)],
                                rows1.at[pl.ds(0, _CH)])
                pltpu.sync_copy(rows1.at[pl.ds(0, _CH)],
                                out_hbm.at[cid, pl.ds(b * _CH, _CH)])

    return sc_kernel


def kernel(x, g, ew):
    N, C = x.shape
    E = ew.shape[0]
    i = g[0].astype(jnp.int32)
    j = g[1].astype(jnp.int32)
    ew = ew.astype(jnp.float32)
    pad = (-E) % _SGE
    if pad:
        i = jnp.concatenate([i, jnp.zeros((pad,), jnp.int32)])
        j = jnp.concatenate([j, jnp.zeros((pad,), jnp.int32)])
        ew = jnp.concatenate([ew, jnp.zeros((pad,), jnp.float32)])
    nblk = (E + pad) // _SGE
    comb = jnp.stack([
        i.reshape(nblk, _SUBS, _CH),
        j.reshape(nblk, _SUBS, _CH),
        lax.bitcast_convert_type(ew, jnp.int32).reshape(nblk, _SUBS, _CH),
    ], axis=1)
    halves = _make_sc_kernel(N, E + pad, C)(x.astype(jnp.float32), comb)
    H = N // 2
    return jnp.concatenate([halves[0, :H], halves[1, :H]], axis=0)


# speedup vs baseline: 1.0005x; 1.0005x over previous
import functools

import jax
import jax.numpy as jnp
from jax import lax
from jax.experimental import pallas as pl
from jax.experimental.pallas import tpu as pltpu
from jax.experimental.pallas import tpu_sc as plsc

_CH = 128
_SUBS = 8
_SGE = _CH * _SUBS
_FIFO = 2 * _SGE
_LANES = 16
_G16 = _SGE // _LANES


@functools.lru_cache(maxsize=None)
def _make_sc_kernel(N, E, C):
    info = plsc.get_sparse_core_info()
    NC, NS = info.num_cores, info.num_subcores
    assert E % _SGE == 0 and C % _LANES == 0 and N % NC == 0
    H = N // NC
    assert H % 8 == 0
    hblk = (H + _CH - 1) // _CH
    h_pad = hblk * _CH
    xfull, xrem = divmod(H, _CH)
    nblk = E // _SGE
    qb, rb = divmod(nblk, NS)
    groups = C // _LANES
    tload = (hblk + NS - 1) // NS

    mesh = plsc.VectorSubcoreMesh(core_axis_name="c", subcore_axis_name="s")

    @functools.partial(
        pl.kernel,
        out_type=jax.ShapeDtypeStruct((NC, h_pad, C), jnp.float32),
        mesh=mesh,
        scratch_types=[
            pltpu.VMEM((_CH, C), jnp.float32),
            pltpu.VMEM((_CH, C), jnp.float32),
            pltpu.VMEM((3, _SUBS, _CH), jnp.int32),
            pltpu.VMEM((_FIFO,), jnp.int32),
            pltpu.VMEM((_FIFO,), jnp.int32),
            pltpu.VMEM((_FIFO,), jnp.int32),
            pltpu.VMEM((_FIFO,), jnp.int32),
            pltpu.VMEM((_FIFO,), jnp.int32),
            pltpu.VMEM((_FIFO,), jnp.int32),
            pltpu.VMEM((_CH,), jnp.int32),
            pltpu.VMEM((_CH,), jnp.int32),
            pltpu.VMEM((2 * _LANES,), jnp.int32),
            pltpu.VMEM_SHARED((H, C), jnp.float32),
            pltpu.VMEM_SHARED((h_pad, C), jnp.float32),
            pltpu.SemaphoreType.DMA,
            pltpu.SemaphoreType.DMA,
            pltpu.SemaphoreType.DMA,
            pltpu.SemaphoreType.DMA,
            pltpu.SemaphoreType.DMA,
        ],
        compiler_params=pltpu.CompilerParams(needs_layout_passes=False),
    )
    def sc_kernel(x_hbm, comb_hbm, out_hbm,
                  rows0, rows1, comb_buf,
                  fli, flj, flw, fhi, fhj, fhw, jrow0, jrow1, tmpc,
                  x_sh, acc,
                  isem, gsem0, gsem1, ssem0, ssem1):
        cid = lax.axis_index("c")
        sid = lax.axis_index("s")
        lo = cid * H

        zero = jnp.zeros((_LANES,), jnp.float32)

        @plsc.parallel_loop(0, _CH, unroll=8)
        def zero_rows(rr):
            for g in range(groups):
                rows0[rr, pl.ds(g * _LANES, _LANES)] = zero

        izero0 = jnp.zeros((_LANES,), jnp.int32)

        @plsc.parallel_loop(0, _FIFO // _LANES, unroll=8)
        def zero_fifos(t):
            o = pl.ds(t * _LANES, _LANES)
            fli[o] = izero0
            flj[o] = izero0
            flw[o] = izero0
            fhi[o] = izero0
            fhj[o] = izero0
            fhw[o] = izero0

        for t in range(tload):
            b = sid + NS * t

            @pl.when(b < hblk)
            def _():
                pltpu.sync_copy(rows0, acc.at[pl.ds(b * _CH, _CH)])

            @pl.when(b < xfull)
            def _():
                pltpu.sync_copy(x_hbm.at[pl.ds(b * _CH, _CH)],
                                x_sh.at[pl.ds(b * _CH, _CH)])

            if xrem:
                @pl.when(b == xfull)
                def _():
                    pltpu.sync_copy(x_hbm.at[pl.ds(xfull * _CH, xrem)],
                                    x_sh.at[pl.ds(xfull * _CH, xrem)])
        plsc.subcore_barrier()

        rows = (rows0, rows1)
        jrows = (jrow0, jrow1)
        gsems = (gsem0, gsem1)
        ssems = (ssem0, ssem1)

        def process_1024(xsrc, fi, fj, fw):
            gd = [None] * _SUBS
            sd = [None] * _SUBS
            gd[0] = pltpu.async_copy(
                xsrc.at[fi.at[pl.ds(0, _CH)]], rows[0], gsems[0])
            for sub in range(_SUBS):
                b = sub & 1
                if sub + 1 < _SUBS:
                    if sub >= 1:
                        sd[sub - 1].wait()
                    gd[sub + 1] = pltpu.async_copy(
                        xsrc.at[fi.at[pl.ds((sub + 1) * _CH, _CH)]],
                        rows[1 - b], gsems[1 - b])
                gd[sub].wait()

                @plsc.parallel_loop(0, _CH, unroll=8)
                def mul_body(e, _rows=rows[b], _sub=sub):
                    w16 = plsc.bitcast(
                        plsc.load_gather(
                            fw, [jnp.full((_LANES,), _sub * _CH, jnp.int32)
                                 + e]),
                        jnp.float32)
                    for g in range(groups):
                        sl = pl.ds(g * _LANES, _LANES)
                        _rows[e, sl] = _rows[e, sl] * w16

                for g in range(_CH // _LANES):
                    jrows[b][pl.ds(g * _LANES, _LANES)] = (
                        fj[pl.ds(sub * _CH + g * _LANES, _LANES)])
                sd[sub] = pltpu.async_copy(
                    rows[b], acc.at[jrows[b]], ssems[b], add=True)
            sd[_SUBS - 2].wait()
            sd[_SUBS - 1].wait()

        bstart = sid * qb + jnp.minimum(sid, rb)
        bcnt = qb + jnp.where(sid < rb, 1, 0)

        izero = jnp.zeros((_LANES,), jnp.int32)

        def blk_body(blk, pos):
            pos_lo, pos_hi = pos
            is_flush = blk >= bcnt
            blk_eff = bstart + jnp.minimum(blk, bcnt - 1)
            pltpu.async_copy(comb_hbm.at[blk_eff], comb_buf, isem).wait()

            def compact(pos2):
                def grp_body(k, p):
                    p_lo, p_hi = p
                    srow = k // (_CH // _LANES)
                    soff = (k % (_CH // _LANES)) * _LANES
                    iv = comb_buf[0, srow, pl.ds(soff, _LANES)]
                    jv = comb_buf[1, srow, pl.ds(soff, _LANES)]
                    wv = comb_buf[2, srow, pl.ds(soff, _LANES)]
                    in_half = (jv >= lo) & (jv < lo + H)
                    is_src_lo = iv < H
                    m_lo = in_half & is_src_lo
                    m_hi = in_half & (~is_src_lo)
                    jl = jv - lo
                    plsc.store_compressed(fli.at[pl.ds(p_lo, _LANES)],
                                          iv, mask=m_lo)
                    plsc.store_compressed(flj.at[pl.ds(p_lo, _LANES)],
                                          jl, mask=m_lo)
                    plsc.store_compressed(flw.at[pl.ds(p_lo, _LANES)],
                                          wv, mask=m_lo)
                    plsc.store_compressed(fhi.at[pl.ds(p_hi, _LANES)],
                                          iv, mask=m_hi)
                    plsc.store_compressed(fhj.at[pl.ds(p_hi, _LANES)],
                                          jl, mask=m_hi)
                    plsc.store_compressed(fhw.at[pl.ds(p_hi, _LANES)],
                                          wv, mask=m_hi)
                    c_lo = plsc.all_reduce_population_count(m_lo)[0]
                    c_hi = plsc.all_reduce_population_count(m_hi)[0]
                    return (p_lo + c_lo, p_hi + c_hi)

                return lax.fori_loop(0, _G16, grp_body, pos2)

            pos_lo, pos_hi = lax.cond(
                is_flush, lambda p: p, compact, (pos_lo, pos_hi))

            @pl.when(is_flush)
            def _():
                for t in range(_SGE // _LANES):
                    fli[pl.ds(pos_lo + t * _LANES, _LANES)] = izero
                    flj[pl.ds(pos_lo + t * _LANES, _LANES)] = izero
                    flw[pl.ds(pos_lo + t * _LANES, _LANES)] = izero
                    fhi[pl.ds(pos_hi + t * _LANES, _LANES)] = izero
                    fhj[pl.ds(pos_hi + t * _LANES, _LANES)] = izero
                    fhw[pl.ds(pos_hi + t * _LANES, _LANES)] = izero

            do_lo = (pos_lo >= _SGE) | (is_flush & (pos_lo > 0))
            do_hi = (pos_hi >= _SGE) | (is_flush & (pos_hi > 0))

            @pl.when(do_lo)
            def _():
                process_1024(x_sh, fli, flj, flw)
                nmv = (jnp.maximum(pos_lo - _SGE, 0) + _LANES - 1) // _LANES

                def mv(t, _):
                    o = t * _LANES
                    fli[pl.ds(o, _LANES)] = fli[pl.ds(_SGE + o, _LANES)]
                    flj[pl.ds(o, _LANES)] = flj[pl.ds(_SGE + o, _LANES)]
                    flw[pl.ds(o, _LANES)] = flw[pl.ds(_SGE + o, _LANES)]
                    return 0

                lax.fori_loop(0, nmv, mv, 0)

            @pl.when(do_hi)
            def _():
                process_1024(x_hbm, fhi, fhj, fhw)
                nmv = (jnp.maximum(pos_hi - _SGE, 0) + _LANES - 1) // _LANES

                def mv(t, _):
                    o = t * _LANES
                    fhi[pl.ds(o, _LANES)] = fhi[pl.ds(_SGE + o, _LANES)]
                    fhj[pl.ds(o, _LANES)] = fhj[pl.ds(_SGE + o, _LANES)]
                    fhw[pl.ds(o, _LANES)] = fhw[pl.ds(_SGE + o, _LANES)]
                    return 0

                lax.fori_loop(0, nmv, mv, 0)

            pos_lo = jnp.where(do_lo, jnp.maximum(pos_lo - _SGE, 0), pos_lo)
            pos_hi = jnp.where(do_hi, jnp.maximum(pos_hi - _SGE, 0), pos_hi)
            return (pos_lo, pos_hi)

        lax.fori_loop(0, bcnt + 1, blk_body,
                      (jnp.int32(0), jnp.int32(0)))
        plsc.subcore_barrier()

        for t in range(tload):
            b = sid + NS * t

            @pl.when(b < hblk)
            def _():
                pltpu.sync_copy(acc.at[pl.ds(b * _CH, _CH)],
                                rows1.at[pl.ds(0, _CH)])
                pltpu.sync_copy(rows1.at[pl.ds(0, _CH)],
                                out_hbm.at[cid, pl.ds(b * _CH, _CH)])

    return sc_kernel


def kernel(x, g, ew):
    N, C = x.shape
    E = ew.shape[0]
    i = g[0].astype(jnp.int32)
    j = g[1].astype(jnp.int32)
    ew = ew.astype(jnp.float32)
    pad = (-E) % _SGE
    if pad:
        i = jnp.concatenate([i, jnp.zeros((pad,), jnp.int32)])
        j = jnp.concatenate([j, jnp.zeros((pad,), jnp.int32)])
        ew = jnp.concatenate([ew, jnp.zeros((pad,), jnp.float32)])
    nblk = (E + pad) // _SGE
    comb = jnp.stack([
        i.reshape(nblk, _SUBS, _CH),
        j.reshape(nblk, _SUBS, _CH),
        lax.bitcast_convert_type(ew, jnp.int32).reshape(nblk, _SUBS, _CH),
    ], axis=1)
    halves = _make_sc_kernel(N, E + pad, C)(x.astype(jnp.float32), comb)
    H = N // 2
    return jnp.concatenate([halves[0, :H], halves[1, :H]], axis=0)


# speedup vs baseline: 3.6855x; 3.6836x over previous
import functools

import jax
import jax.numpy as jnp
from jax import lax
from jax.experimental import pallas as pl
from jax.experimental.pallas import tpu as pltpu
from jax.experimental.pallas import tpu_sc as plsc

_CH = 128
_SUBS = 8
_SGE = _CH * _SUBS
_LANES = 16


@functools.lru_cache(maxsize=None)
def _make_sc_kernel(N, E, C):
    info = plsc.get_sparse_core_info()
    NC, NS = info.num_cores, info.num_subcores
    assert E % _SGE == 0 and C % _LANES == 0
    NW = NC * NS
    nsg = E // _SGE
    q, r = divmod(nsg, NW)
    rows_per_tile = ((N + NS - 1) // NS + 7) // 8 * 8
    n_pad = NS * rows_per_tile
    groups = C // _LANES

    mesh = plsc.VectorSubcoreMesh(core_axis_name="c", subcore_axis_name="s")

    @functools.partial(
        pl.kernel,
        out_type=jax.ShapeDtypeStruct((NC, n_pad, C), jnp.float32),
        mesh=mesh,
        scratch_types=[
            pltpu.VMEM((_CH, C), jnp.float32),
            pltpu.VMEM((_CH, C), jnp.float32),
            pltpu.VMEM((3, _SUBS, _CH), jnp.int32),
            pltpu.VMEM_SHARED((n_pad, C), jnp.float32),
            pltpu.SemaphoreType.DMA,
            pltpu.SemaphoreType.DMA,
            pltpu.SemaphoreType.DMA,
            pltpu.SemaphoreType.DMA,
            pltpu.SemaphoreType.DMA,
        ],
        compiler_params=pltpu.CompilerParams(needs_layout_passes=False),
    )
    def sc_kernel(x_hbm, comb_hbm, out_hbm,
                  rows0, rows1, comb_buf, acc,
                  isem, gsem0, gsem1, ssem0, ssem1):
        cid = lax.axis_index("c")
        sid = lax.axis_index("s")
        wid = sid * NC + cid

        zero = jnp.zeros((_LANES,), jnp.float32)

        @plsc.parallel_loop(0, _CH, unroll=8)
        def zero_rows(rr):
            for g in range(groups):
                rows0[rr, pl.ds(g * _LANES, _LANES)] = zero

        row0 = sid * rows_per_tile
        for m in range(0, rows_per_tile, _CH):
            sz = min(_CH, rows_per_tile - m)
            pltpu.sync_copy(rows0.at[pl.ds(0, sz)],
                            acc.at[pl.ds(row0 + m, sz)])
        plsc.subcore_barrier()

        start = wid * q + jnp.minimum(wid, r)
        cnt = q + jnp.where(wid < r, 1, 0)

        rows = (rows0, rows1)
        gsems = (gsem0, gsem1)
        ssems = (ssem0, ssem1)

        def sg_body(sg, _):
            pltpu.async_copy(comb_hbm.at[start + sg], comb_buf, isem).wait()

            gd = [None] * _SUBS
            sd = [None] * _SUBS
            gd[0] = pltpu.async_copy(
                x_hbm.at[comb_buf.at[0, 0]], rows[0], gsems[0])
            for sub in range(_SUBS):
                b = sub & 1
                if sub + 1 < _SUBS:
                    if sub >= 1:
                        sd[sub - 1].wait()
                    gd[sub + 1] = pltpu.async_copy(
                        x_hbm.at[comb_buf.at[0, sub + 1]], rows[1 - b],
                        gsems[1 - b])
                gd[sub].wait()

                @plsc.parallel_loop(0, _CH, unroll=8)
                def mul_body(e, _rows=rows[b], _sub=sub):
                    w16 = plsc.bitcast(
                        plsc.load_gather(
                            comb_buf,
                            [jnp.full((_LANES,), 2, jnp.int32),
                             jnp.full((_LANES,), _sub, jnp.int32),
                             jnp.full((_LANES,), e, jnp.int32)]),
                        jnp.float32)
                    for g in range(groups):
                        sl = pl.ds(g * _LANES, _LANES)
                        _rows[e, sl] = _rows[e, sl] * w16

                sd[sub] = pltpu.async_copy(
                    rows[b], acc.at[comb_buf.at[1, sub]], ssems[b], add=True)
            sd[_SUBS - 2].wait()
            sd[_SUBS - 1].wait()
            return 0

        lax.fori_loop(0, cnt, sg_body, 0)
        plsc.subcore_barrier()

        for m in range(0, rows_per_tile, _CH):
            sz = min(_CH, rows_per_tile - m)
            pltpu.sync_copy(acc.at[pl.ds(row0 + m, sz)],
                            rows0.at[pl.ds(0, sz)])
            pltpu.sync_copy(rows0.at[pl.ds(0, sz)],
                            out_hbm.at[cid, pl.ds(row0 + m, sz)])

    return sc_kernel


def _tc_add(partials, N, C):
    blk = 1000

    def add_body(p_ref, o_ref):
        o_ref[...] = p_ref[0] + p_ref[1]

    return pl.pallas_call(
        add_body,
        out_shape=jax.ShapeDtypeStruct((N, C), jnp.float32),
        grid=(N // blk,),
        in_specs=[pl.BlockSpec((2, blk, C), lambda i: (0, i, 0))],
        out_specs=pl.BlockSpec((blk, C), lambda i: (i, 0)),
    )(partials)


def kernel(x, g, ew):
    N, C = x.shape
    E = ew.shape[0]
    i = g[0].astype(jnp.int32)
    j = g[1].astype(jnp.int32)
    ew = ew.astype(jnp.float32)
    pad = (-E) % _SGE
    if pad:
        i = jnp.concatenate([i, jnp.zeros((pad,), jnp.int32)])
        j = jnp.concatenate([j, jnp.zeros((pad,), jnp.int32)])
        ew = jnp.concatenate([ew, jnp.zeros((pad,), jnp.float32)])
    nsg = (E + pad) // _SGE
    comb = jnp.stack([
        i.reshape(nsg, _SUBS, _CH),
        j.reshape(nsg, _SUBS, _CH),
        lax.bitcast_convert_type(ew, jnp.int32).reshape(nsg, _SUBS, _CH),
    ], axis=1)
    partials = _make_sc_kernel(N, E + pad, C)(x.astype(jnp.float32), comb)
    return _tc_add(partials, N, C)
